# Initial kernel scaffold; baseline (speedup 1.0000x reference)
import jax
import jax.numpy as jnp
from jax.experimental import pallas as pl


def kernel(features, edge_index, W):
    raise NotImplementedError("write your pallas kernel here")


# speedup vs baseline: 4.8079x; 4.8079x over previous
import functools

import jax
import jax.numpy as jnp
from jax import lax
from jax.experimental import pallas as pl
from jax.experimental.pallas import tpu as pltpu
from jax.experimental.pallas import tpu_sc as plsc

N_NODES = 10000
IN_FEAT = 128
OUT_FEAT = 128

NPAD = 10240
NW = 32
ROWS_PER_TILE = NPAD // 16
CHUNK = 128


def _sc_body(feat_hbm, src_hbm, dst_hbm, seg_out, cnt_out,
             src_v, dst_v, rows_v, ones_v, zc_v, seg_sh, cnt_sh, sem,
             *, chunks_per_tile):
    cid = lax.axis_index("c")
    sid = lax.axis_index("s")
    wid = sid * 2 + cid

    zrow = jnp.zeros((16,), jnp.float32)

    def zero_rows(i, _):
        for j in range(IN_FEAT // 16):
            rows_v[i, pl.ds(j * 16, 16)] = zrow
        return 0
    lax.fori_loop(0, CHUNK, zero_rows, 0)

    def zero_zc(i, _):
        zc_v[pl.ds(i * 16, 16)] = zrow
        return 0
    lax.fori_loop(0, ROWS_PER_TILE // 16, zero_zc, 0)

    for j in range(CHUNK // 16):
        ones_v[pl.ds(j * 16, 16)] = jnp.ones((16,), jnp.float32)

    base_n = sid * ROWS_PER_TILE
    for t in range(ROWS_PER_TILE // CHUNK):
        pltpu.sync_copy(rows_v, seg_sh.at[pl.ds(base_n + t * CHUNK, CHUNK)])
    pltpu.sync_copy(zc_v, cnt_sh.at[pl.ds(base_n, ROWS_PER_TILE)])

    plsc.subcore_barrier()

    base_e = wid * (chunks_per_tile * CHUNK)

    def edge_step(t, _):
        off = base_e + t * CHUNK
        pltpu.sync_copy(src_hbm.at[pl.ds(off, CHUNK)], src_v)
        pltpu.sync_copy(dst_hbm.at[pl.ds(off, CHUNK)], dst_v)
        pltpu.async_copy(feat_hbm.at[src_v], rows_v, sem).wait()
        pltpu.sync_copy(rows_v, seg_sh.at[dst_v], add=True)
        pltpu.sync_copy(ones_v, cnt_sh.at[dst_v], add=True)
        return 0
    lax.fori_loop(0, chunks_per_tile, edge_step, 0)

    plsc.subcore_barrier()

    pltpu.sync_copy(seg_sh.at[pl.ds(base_n, ROWS_PER_TILE)],
                    seg_out.at[cid, pl.ds(base_n, ROWS_PER_TILE)])
    pltpu.sync_copy(cnt_sh.at[pl.ds(base_n, ROWS_PER_TILE)],
                    cnt_out.at[cid, pl.ds(base_n, ROWS_PER_TILE)])


def _segment_sum_sc(features, src, dst, chunks_per_tile):
    mesh = plsc.VectorSubcoreMesh(core_axis_name="c", subcore_axis_name="s")
    body = functools.partial(_sc_body, chunks_per_tile=chunks_per_tile)
    return pl.kernel(
        body,
        out_type=[
            jax.ShapeDtypeStruct((2, NPAD, IN_FEAT), jnp.float32),
            jax.ShapeDtypeStruct((2, NPAD), jnp.float32),
        ],
        mesh=mesh,
        scratch_types=[
            pltpu.VMEM((CHUNK,), jnp.int32),
            pltpu.VMEM((CHUNK,), jnp.int32),
            pltpu.VMEM((CHUNK, IN_FEAT), jnp.float32),
            pltpu.VMEM((CHUNK,), jnp.float32),
            pltpu.VMEM((ROWS_PER_TILE,), jnp.float32),
            pltpu.VMEM_SHARED((NPAD, IN_FEAT), jnp.float32),
            pltpu.VMEM_SHARED((NPAD,), jnp.float32),
            pltpu.SemaphoreType.DMA,
        ],
    )(features, src, dst)


def _tc_body(feat_ref, w_ref, seg_ref, cnt_ref, out_ref):
    i = pl.program_id(0)
    blk = feat_ref.shape[0]
    w = w_ref[...]
    nodes_rep = jnp.dot(feat_ref[...], w, preferred_element_type=jnp.float32)
    seg = seg_ref[0] + seg_ref[1]
    cnt = cnt_ref[0, pl.ds(i * blk, blk)] + cnt_ref[1, pl.ds(i * blk, blk)]
    agg = seg / jnp.maximum(cnt, 1.0)[:, None]
    msgs = jnp.dot(agg, w, preferred_element_type=jnp.float32)
    out_ref[:, :OUT_FEAT] = jnp.maximum(nodes_rep, 0.0)
    out_ref[:, OUT_FEAT:] = jnp.maximum(msgs, 0.0)


def _combine_tc(feat_pad, W, seg_p, cnt_p):
    blk = 1024
    grid = (NPAD // blk,)
    return pl.pallas_call(
        _tc_body,
        grid=grid,
        in_specs=[
            pl.BlockSpec((blk, IN_FEAT), lambda i: (i, 0)),
            pl.BlockSpec((IN_FEAT, OUT_FEAT), lambda i: (0, 0)),
            pl.BlockSpec((2, blk, IN_FEAT), lambda i: (0, i, 0)),
            pl.BlockSpec((2, NPAD), lambda i: (0, 0)),
        ],
        out_specs=pl.BlockSpec((blk, 2 * OUT_FEAT), lambda i: (i, 0)),
        out_shape=jax.ShapeDtypeStruct((NPAD, 2 * OUT_FEAT), jnp.float32),
    )(feat_pad, W, seg_p, cnt_p)


def kernel(features, edge_index, W):
    n_edges = edge_index.shape[1]
    ept = -(-n_edges // (NW * CHUNK)) * CHUNK
    epad = ept * NW
    ei = edge_index.astype(jnp.int32)
    pad = epad - n_edges
    src = jnp.concatenate([ei[1], jnp.zeros((pad,), jnp.int32)])
    dst = jnp.concatenate([ei[0], jnp.full((pad,), N_NODES, jnp.int32)])

    seg_p, cnt_p = _segment_sum_sc(features, src, dst, ept // CHUNK)

    feat_pad = jnp.pad(features, ((0, NPAD - N_NODES), (0, 0)))
    out = _combine_tc(feat_pad, W, seg_p, cnt_p)
    return out[:N_NODES]
